# baseline (speedup 1.0000x reference)
import functools

import jax
import jax.numpy as jnp
from jax import lax
from jax.experimental import pallas as pl
from jax.experimental.pallas import tpu as pltpu
from jax.experimental.pallas import tpu_sc as plsc

N = 10000
NP = 10240
D = 128
E = 320000
NUM_TILES = 32
CHUNK = 125
CHUNKS_PER_TILE = 80
EDGES_PER_TILE = CHUNK * CHUNKS_PER_TILE
ROWS_PER_TILE = NP // 16
ROWS_PER_BLOCK = 512
GRID = NP // ROWS_PER_BLOCK
SUB = ROWS_PER_BLOCK // 128
GROUP = 16
NUM_GROUPS = CHUNKS_PER_TILE // GROUP


_MESH = plsc.VectorSubcoreMesh(core_axis_name="c", subcore_axis_name="s")


@functools.partial(
    pl.kernel,
    out_type=jax.ShapeDtypeStruct((2, NP), jnp.float32),
    mesh=_MESH,
    scratch_types=[
        pltpu.VMEM_SHARED((NP,), jnp.float32),
        pltpu.VMEM((CHUNKS_PER_TILE, CHUNK), jnp.int32),
        pltpu.VMEM((CHUNK,), jnp.float32),
        pltpu.VMEM((ROWS_PER_TILE,), jnp.float32),
    ],
)
def _sc_degree(dst_hbm, out_hbm, deg_sh, dst_v, ones_v, zero_v):
    c = lax.axis_index("c")
    s = lax.axis_index("s")
    wid = c * 16 + s

    zvec = jnp.zeros((16,), jnp.float32)
    ovec = jnp.ones((16,), jnp.float32)

    def fill(i, carry):
        zero_v[pl.ds(i * 16, 16)] = zvec
        return carry

    lax.fori_loop(0, ROWS_PER_TILE // 16, fill, 0)
    for k in range(CHUNK // 16):
        ones_v[pl.ds(k * 16, 16)] = ovec
    ones_v[pl.ds(CHUNK - 16, 16)] = ovec

    pltpu.sync_copy(zero_v, deg_sh.at[pl.ds(s * ROWS_PER_TILE, ROWS_PER_TILE)])
    pltpu.sync_copy(
        dst_hbm.at[pl.ds(wid * CHUNKS_PER_TILE, CHUNKS_PER_TILE)], dst_v)
    plsc.subcore_barrier()

    def body(j, carry):
        pltpu.sync_copy(ones_v, deg_sh.at[dst_v.at[j]], add=True)
        return carry

    lax.fori_loop(0, CHUNKS_PER_TILE, body, 0)
    plsc.subcore_barrier()

    @pl.when(s == 0)
    def _flush():
        pltpu.sync_copy(deg_sh, out_hbm.at[c])


@functools.partial(
    pl.kernel,
    out_type=jax.ShapeDtypeStruct((2, NP, D), jnp.float32),
    mesh=_MESH,
    scratch_types=[
        pltpu.VMEM_SHARED((NP, D), jnp.float32),
        pltpu.VMEM((2, GROUP, CHUNK), jnp.int32),
        pltpu.VMEM((2, GROUP, CHUNK), jnp.int32),
        pltpu.VMEM((CHUNK, D), jnp.float32),
        pltpu.VMEM((CHUNK, D), jnp.float32),
        pltpu.SemaphoreType.DMA,
        pltpu.SemaphoreType.DMA,
    ],
)
def _sc_scatter(hp_hbm, ei_hbm, out_hbm,
                acc_sh, src_v, dst_v, buf_a, buf_b, sem_a, sem_b):
    c = lax.axis_index("c")
    s = lax.axis_index("s")
    wid = c * 16 + s
    base = wid * CHUNKS_PER_TILE

    pltpu.sync_copy(ei_hbm.at[0, pl.ds(base, GROUP)], src_v.at[0])
    pltpu.sync_copy(ei_hbm.at[1, pl.ds(base, GROUP)], dst_v.at[0])

    pltpu.async_copy(hp_hbm.at[src_v.at[0, 0]], buf_a, sem_a)

    zvec = jnp.zeros((16,), jnp.float32)

    def zfill(i, carry):
        for k in range(D // 16):
            buf_b[i, pl.ds(k * 16, 16)] = zvec
        return carry

    lax.fori_loop(0, CHUNK, zfill, 0)
    row0 = s * ROWS_PER_TILE
    for r in range(ROWS_PER_TILE // CHUNK):
        pltpu.sync_copy(buf_b, acc_sh.at[pl.ds(row0 + r * CHUNK, CHUNK)])
    rem = ROWS_PER_TILE % CHUNK
    if rem:
        pltpu.sync_copy(buf_b.at[pl.ds(0, rem)],
                        acc_sh.at[pl.ds(row0 + ROWS_PER_TILE - rem, rem)])

    pltpu.async_copy(hp_hbm.at[src_v.at[0, 1]], buf_b, sem_b)
    plsc.subcore_barrier()

    def step(j, buf, sem):
        g = j // GROUP
        k = j % GROUP
        slot = g % 2

        @pl.when(jnp.logical_and(k == 0, g + 1 < NUM_GROUPS))
        def _prefetch_group():
            nxt = g + 1
            pltpu.sync_copy(ei_hbm.at[0, pl.ds(base + nxt * GROUP, GROUP)],
                            src_v.at[nxt % 2])
            pltpu.sync_copy(ei_hbm.at[1, pl.ds(base + nxt * GROUP, GROUP)],
                            dst_v.at[nxt % 2])

        pltpu.make_async_copy(hp_hbm.at[src_v.at[slot, k]], buf, sem).wait()
        pltpu.sync_copy(buf, acc_sh.at[dst_v.at[slot, k]], add=True)

        @pl.when(j + 2 < CHUNKS_PER_TILE)
        def _next():
            jn = j + 2
            pltpu.async_copy(
                hp_hbm.at[src_v.at[(jn // GROUP) % 2, jn % GROUP]], buf, sem)

    def body(i, carry):
        step(2 * i, buf_a, sem_a)
        step(2 * i + 1, buf_b, sem_b)
        return carry

    lax.fori_loop(0, CHUNKS_PER_TILE // 2, body, 0)
    plsc.subcore_barrier()

    @pl.when(s == 0)
    def _flush():
        pltpu.sync_copy(acc_sh, out_hbm.at[c])


def _dinv_col(deg_blk):
    deg = deg_blk[0, 0] + deg_blk[1, 0] + 1.0
    dinv = lax.rsqrt(deg)
    r_sub = lax.broadcasted_iota(jnp.int32, (ROWS_PER_BLOCK, SUB), 0) // 128
    k_sub = lax.broadcasted_iota(jnp.int32, (ROWS_PER_BLOCK, SUB), 1)
    onehot = (r_sub == k_sub).astype(jnp.float32)
    rows = jnp.dot(onehot, dinv, preferred_element_type=jnp.float32)
    r_lane = lax.broadcasted_iota(jnp.int32, (ROWS_PER_BLOCK, 128), 0) % 128
    m_lane = lax.broadcasted_iota(jnp.int32, (ROWS_PER_BLOCK, 128), 1)
    sel = (r_lane == m_lane).astype(jnp.float32)
    return jnp.sum(rows * sel, axis=1, keepdims=True)


def _tc1_body(x_ref, w_ref, deg_ref, hp_ref):
    h = jnp.dot(x_ref[...], w_ref[...], preferred_element_type=jnp.float32)
    hp_ref[...] = h * _dinv_col(deg_ref[...])


def _tc2_body(acc_ref, hp_ref, deg_ref, w_ref, b_ref, hp2_ref):
    dinv = _dinv_col(deg_ref[...])
    out1 = (acc_ref[0] + acc_ref[1] + hp_ref[...]) * dinv + b_ref[...]
    h2 = jnp.dot(out1, w_ref[...], preferred_element_type=jnp.float32)
    hp2_ref[...] = h2 * dinv


def _tc3_body(acc_ref, hp_ref, deg_ref, b_ref, out_ref):
    dinv = _dinv_col(deg_ref[...])
    out_ref[...] = (acc_ref[0] + acc_ref[1] + hp_ref[...]) * dinv + b_ref[...]


_row_spec = pl.BlockSpec((ROWS_PER_BLOCK, D), lambda i: (i, 0))
_w_spec = pl.BlockSpec((D, D), lambda i: (0, 0))
_b_spec = pl.BlockSpec((1, D), lambda i: (0, 0))
_deg_spec = pl.BlockSpec((2, 1, SUB, D), lambda i: (0, i, 0, 0))
_acc_spec = pl.BlockSpec((2, ROWS_PER_BLOCK, D), lambda i: (0, i, 0))

_tc1 = pl.pallas_call(
    _tc1_body,
    grid=(GRID,),
    in_specs=[_row_spec, _w_spec, _deg_spec],
    out_specs=_row_spec,
    out_shape=jax.ShapeDtypeStruct((NP, D), jnp.float32),
)

_tc2 = pl.pallas_call(
    _tc2_body,
    grid=(GRID,),
    in_specs=[_acc_spec, _row_spec, _deg_spec, _w_spec, _b_spec],
    out_specs=_row_spec,
    out_shape=jax.ShapeDtypeStruct((NP, D), jnp.float32),
)

_tc3 = pl.pallas_call(
    _tc3_body,
    grid=(GRID,),
    in_specs=[_acc_spec, _row_spec, _deg_spec, _b_spec],
    out_specs=_row_spec,
    out_shape=jax.ShapeDtypeStruct((N, D), jnp.float32),
)


def kernel(x, edge_index, W1, b1, W2, b2):
    ei = edge_index.astype(jnp.int32).reshape(
        2, NUM_TILES * CHUNKS_PER_TILE, CHUNK)
    b1r = b1.reshape(1, D)
    b2r = b2.reshape(1, D)

    dst2 = edge_index[1].astype(jnp.int32).reshape(
        NUM_TILES * CHUNKS_PER_TILE, CHUNK)
    deg2 = _sc_degree(dst2).reshape(2, GRID, SUB, 128)

    hp1 = _tc1(x, W1, deg2)
    acc1 = _sc_scatter(hp1, ei)
    hp2 = _tc2(acc1, hp1, deg2, W2, b1r)
    acc2 = _sc_scatter(hp2, ei)
    return _tc3(acc2, hp2, deg2, b2r)


# speedup vs baseline: 1.0934x; 1.0934x over previous
import functools

import jax
import jax.numpy as jnp
from jax import lax
from jax.experimental import pallas as pl
from jax.experimental.pallas import tpu as pltpu
from jax.experimental.pallas import tpu_sc as plsc

N = 10000
NP = 10240
D = 128
E = 320000
NUM_TILES = 32
CHUNK = 125
CHUNKS_PER_TILE = 80
EDGES_PER_TILE = CHUNK * CHUNKS_PER_TILE
ROWS_PER_TILE = NP // 16
ROWS_PER_BLOCK = 1024
GRID = NP // ROWS_PER_BLOCK
GROUP = 16
NUM_GROUPS = CHUNKS_PER_TILE // GROUP


_MESH = plsc.VectorSubcoreMesh(core_axis_name="c", subcore_axis_name="s")


@functools.partial(
    pl.kernel,
    out_type=jax.ShapeDtypeStruct((2, NP), jnp.float32),
    mesh=_MESH,
    scratch_types=[
        pltpu.VMEM_SHARED((NP,), jnp.float32),
        pltpu.VMEM((CHUNKS_PER_TILE, CHUNK), jnp.int32),
        pltpu.VMEM((CHUNK,), jnp.float32),
        pltpu.VMEM((ROWS_PER_TILE,), jnp.float32),
    ],
)
def _sc_degree(ei_hbm, out_hbm, deg_sh, dst_v, ones_v, zero_v):
    c = lax.axis_index("c")
    s = lax.axis_index("s")
    wid = c * 16 + s

    zvec = jnp.zeros((16,), jnp.float32)
    ovec = jnp.ones((16,), jnp.float32)

    def fill(i, carry):
        zero_v[pl.ds(i * 16, 16)] = zvec
        return carry

    lax.fori_loop(0, ROWS_PER_TILE // 16, fill, 0)
    for k in range(CHUNK // 16):
        ones_v[pl.ds(k * 16, 16)] = ovec
    ones_v[pl.ds(CHUNK - 16, 16)] = ovec

    pltpu.sync_copy(zero_v, deg_sh.at[pl.ds(s * ROWS_PER_TILE, ROWS_PER_TILE)])
    pltpu.sync_copy(
        ei_hbm.at[1, pl.ds(wid * CHUNKS_PER_TILE, CHUNKS_PER_TILE)], dst_v)
    plsc.subcore_barrier()

    def body(j, carry):
        pltpu.sync_copy(ones_v, deg_sh.at[dst_v.at[j]], add=True)
        return carry

    lax.fori_loop(0, CHUNKS_PER_TILE, body, 0)
    plsc.subcore_barrier()

    @pl.when(s == 0)
    def _flush():
        pltpu.sync_copy(deg_sh, out_hbm.at[c])


@functools.partial(
    pl.kernel,
    out_type=jax.ShapeDtypeStruct((2, NP, D), jnp.float32),
    mesh=_MESH,
    scratch_types=[
        pltpu.VMEM_SHARED((NP, D), jnp.float32),
        pltpu.VMEM((2, GROUP, CHUNK), jnp.int32),
        pltpu.VMEM((2, GROUP, CHUNK), jnp.int32),
        pltpu.VMEM((CHUNK, D), jnp.float32),
        pltpu.VMEM((CHUNK, D), jnp.float32),
        pltpu.SemaphoreType.DMA,
        pltpu.SemaphoreType.DMA,
    ],
)
def _sc_scatter(hp_hbm, ei_hbm, out_hbm,
                acc_sh, src_v, dst_v, buf_a, buf_b, sem_a, sem_b):
    c = lax.axis_index("c")
    s = lax.axis_index("s")
    wid = c * 16 + s
    base = wid * CHUNKS_PER_TILE

    pltpu.sync_copy(ei_hbm.at[0, pl.ds(base, GROUP)], src_v.at[0])
    pltpu.sync_copy(ei_hbm.at[1, pl.ds(base, GROUP)], dst_v.at[0])

    pltpu.async_copy(hp_hbm.at[src_v.at[0, 0]], buf_a, sem_a)

    zvec = jnp.zeros((16,), jnp.float32)

    def zfill(i, carry):
        for k in range(D // 16):
            buf_b[i, pl.ds(k * 16, 16)] = zvec
        return carry

    lax.fori_loop(0, CHUNK, zfill, 0)
    row0 = s * ROWS_PER_TILE
    for r in range(ROWS_PER_TILE // CHUNK):
        pltpu.sync_copy(buf_b, acc_sh.at[pl.ds(row0 + r * CHUNK, CHUNK)])
    rem = ROWS_PER_TILE % CHUNK
    if rem:
        pltpu.sync_copy(buf_b.at[pl.ds(0, rem)],
                        acc_sh.at[pl.ds(row0 + ROWS_PER_TILE - rem, rem)])

    pltpu.async_copy(hp_hbm.at[src_v.at[0, 1]], buf_b, sem_b)
    plsc.subcore_barrier()

    def step(j, buf, sem):
        g = j // GROUP
        k = j % GROUP
        slot = g % 2

        @pl.when(jnp.logical_and(k == 0, g + 1 < NUM_GROUPS))
        def _prefetch_group():
            nxt = g + 1
            pltpu.sync_copy(ei_hbm.at[0, pl.ds(base + nxt * GROUP, GROUP)],
                            src_v.at[nxt % 2])
            pltpu.sync_copy(ei_hbm.at[1, pl.ds(base + nxt * GROUP, GROUP)],
                            dst_v.at[nxt % 2])

        pltpu.make_async_copy(hp_hbm.at[src_v.at[slot, k]], buf, sem).wait()
        pltpu.sync_copy(buf, acc_sh.at[dst_v.at[slot, k]], add=True)

        @pl.when(j + 2 < CHUNKS_PER_TILE)
        def _next():
            jn = j + 2
            pltpu.async_copy(
                hp_hbm.at[src_v.at[(jn // GROUP) % 2, jn % GROUP]], buf, sem)

    def body(i, carry):
        step(2 * i, buf_a, sem_a)
        step(2 * i + 1, buf_b, sem_b)
        return carry

    lax.fori_loop(0, CHUNKS_PER_TILE // 2, body, 0)
    plsc.subcore_barrier()

    @pl.when(s == 0)
    def _flush():
        pltpu.sync_copy(acc_sh, out_hbm.at[c])


def _dinv_col(deg_blk):
    deg = deg_blk[0] + deg_blk[1] + 1.0
    dinv = lax.rsqrt(deg)
    r_sub = lax.broadcasted_iota(jnp.int32, (ROWS_PER_BLOCK, 8), 0) // 128
    k_sub = lax.broadcasted_iota(jnp.int32, (ROWS_PER_BLOCK, 8), 1)
    onehot = (r_sub == k_sub).astype(jnp.float32)
    rows = jnp.dot(onehot, dinv, preferred_element_type=jnp.float32)
    r_lane = lax.broadcasted_iota(jnp.int32, (ROWS_PER_BLOCK, 128), 0) % 128
    m_lane = lax.broadcasted_iota(jnp.int32, (ROWS_PER_BLOCK, 128), 1)
    sel = (r_lane == m_lane).astype(jnp.float32)
    return jnp.sum(rows * sel, axis=1, keepdims=True)


def _tc1_body(x_ref, w_ref, deg_ref, hp_ref):
    h = jnp.dot(x_ref[...], w_ref[...], preferred_element_type=jnp.float32)
    hp_ref[...] = h * _dinv_col(deg_ref[...])


def _tc2_body(acc_ref, hp_ref, deg_ref, w_ref, b_ref, hp2_ref):
    dinv = _dinv_col(deg_ref[...])
    out1 = (acc_ref[0] + acc_ref[1] + hp_ref[...]) * dinv + b_ref[...]
    h2 = jnp.dot(out1, w_ref[...], preferred_element_type=jnp.float32)
    hp2_ref[...] = h2 * dinv


def _tc3_body(acc_ref, hp_ref, deg_ref, b_ref, out_ref):
    dinv = _dinv_col(deg_ref[...])
    out_ref[...] = (acc_ref[0] + acc_ref[1] + hp_ref[...]) * dinv + b_ref[...]


_row_spec = pl.BlockSpec((ROWS_PER_BLOCK, D), lambda i: (i, 0))
_w_spec = pl.BlockSpec((D, D), lambda i: (0, 0))
_b_spec = pl.BlockSpec((1, D), lambda i: (0, 0))
_deg_spec = pl.BlockSpec((2, 8, D), lambda i: (0, i, 0))
_acc_spec = pl.BlockSpec((2, ROWS_PER_BLOCK, D), lambda i: (0, i, 0))

_tc1 = pl.pallas_call(
    _tc1_body,
    grid=(GRID,),
    in_specs=[_row_spec, _w_spec, _deg_spec],
    out_specs=_row_spec,
    out_shape=jax.ShapeDtypeStruct((NP, D), jnp.float32),
)

_tc2 = pl.pallas_call(
    _tc2_body,
    grid=(GRID,),
    in_specs=[_acc_spec, _row_spec, _deg_spec, _w_spec, _b_spec],
    out_specs=_row_spec,
    out_shape=jax.ShapeDtypeStruct((NP, D), jnp.float32),
)

_tc3 = pl.pallas_call(
    _tc3_body,
    grid=(GRID,),
    in_specs=[_acc_spec, _row_spec, _deg_spec, _b_spec],
    out_specs=_row_spec,
    out_shape=jax.ShapeDtypeStruct((N, D), jnp.float32),
)


def kernel(x, edge_index, W1, b1, W2, b2):
    ei = edge_index.astype(jnp.int32).reshape(
        2, NUM_TILES * CHUNKS_PER_TILE, CHUNK)
    b1r = b1.reshape(1, D)
    b2r = b2.reshape(1, D)

    deg2 = _sc_degree(ei).reshape(2, NP // 128, 128)

    hp1 = _tc1(x, W1, deg2)
    acc1 = _sc_scatter(hp1, ei)
    hp2 = _tc2(acc1, hp1, deg2, W2, b1r)
    acc2 = _sc_scatter(hp2, ei)
    return _tc3(acc2, hp2, deg2, b2r)
